# baseline (speedup 1.0000x reference)
import jax
import jax.numpy as jnp
from jax import lax
from jax.experimental import pallas as pl
from jax.experimental.pallas import tpu as pltpu
from jax.experimental.pallas import tpu_sc as plsc

KMER_NUM = 100000
CLASS_NUM = 100
DIM = 32
B = 1024
L = 200

NC = 2
NS = 16
NW = NC * NS
B_PER_W = B // NW
IDX_COLS = 128
HALF = 16
PK = DIM // 2
EMB_COLS = 128
EMB_ROWS = KMER_NUM * DIM // EMB_COLS
RPF = EMB_COLS // DIM
PCW = 25
PCL = PCW * RPF
N_CHUNKS_TOTAL = EMB_ROWS // PCW
NPC = -(-N_CHUNKS_TOTAL // NS)


def _sc_body(embs_hbm, idx_hbm, out_hbm, tab_sh, idx_v,
             fbuf0, fbuf1, pbuf0, pbuf1, gbuf0, gbuf1, acc_v,
             psem0, psem1, gsem0, gsem1, osem0, osem1):
    cid = lax.axis_index("c")
    sid = lax.axis_index("s")
    wid = sid * NC + cid

    pltpu.sync_copy(idx_hbm.at[pl.ds(wid * B_PER_W * L, B_PER_W * L)], idx_v)

    fbufs, pbufs, psems = (fbuf0, fbuf1), (pbuf0, pbuf1), (psem0, psem1)
    osems = (osem0, osem1)
    c8000 = jnp.full((HALF,), 0x8000, jnp.int32)
    chi = jnp.full((HALF,), -65536, jnp.int32)

    def pack_chunk(slot):
        fbuf, pbuf = fbufs[slot], pbufs[slot]

        def body(i, carry):
            for g in range(RPF):
                u1 = plsc.bitcast(fbuf[i, g * DIM:g * DIM + HALF], jnp.int32)
                u2 = plsc.bitcast(fbuf[i, g * DIM + HALF:(g + 1) * DIM],
                                  jnp.int32)
                lo = lax.shift_right_logical(u1 + c8000, 16)
                hi = lax.bitwise_and(u2 + c8000, chi)
                pbuf[i * RPF + g, 0:PK] = lax.bitwise_or(lo, hi)
            return carry

        lax.fori_loop(0, PCW, body, 0)

    def fire_in(i, slot):
        c = i * NS + sid
        return pltpu.async_copy(embs_hbm.at[pl.ds(c * PCW, PCW)],
                                fbufs[slot], psems[slot])

    N_FULL = N_CHUNKS_TOTAL // NS

    with jax.named_scope("pack_phase"):
        inflight = fire_in(0, 0)
        out_inflight = [None, None]
        for i in range(N_FULL):
            slot = i % 2
            cur = inflight
            if i + 1 < N_FULL:
                inflight = fire_in(i + 1, (i + 1) % 2)
            cur.wait()
            if out_inflight[slot] is not None:
                out_inflight[slot].wait()
            pack_chunk(slot)
            c = i * NS + sid
            out_inflight[slot] = pltpu.async_copy(
                pbufs[slot], tab_sh.at[pl.ds(c * PCL, PCL)], osems[slot])
        for h in out_inflight:
            h.wait()

        @pl.when(N_FULL * NS + sid < N_CHUNKS_TOTAL)
        def _tail():
            c = N_FULL * NS + sid
            pltpu.sync_copy(embs_hbm.at[pl.ds(c * PCW, PCW)], fbufs[0])
            pack_chunk(0)
            pltpu.sync_copy(pbufs[0], tab_sh.at[pl.ds(c * PCL, PCL)])

        plsc.subcore_barrier()

    gbufs, gsems = (gbuf0, gbuf1), (gsem0, gsem1)
    gather_scope = jax.named_scope("gather_phase")
    gather_scope.__enter__()

    def fire(r, slot):
        h0 = pltpu.async_copy(
            tab_sh.at[idx_v.at[pl.ds(r * L, IDX_COLS)]],
            gbufs[slot].at[pl.ds(0, IDX_COLS)], gsems[slot])
        h1 = pltpu.async_copy(
            tab_sh.at[idx_v.at[pl.ds(r * L + IDX_COLS, L - IDX_COLS)]],
            gbufs[slot].at[pl.ds(IDX_COLS, L - IDX_COLS)], gsems[slot])
        return [h0, h1]

    inflight = fire(0, 0)
    for r in range(B_PER_W):
        slot = r % 2
        cur = inflight
        if r + 1 < B_PER_W:
            inflight = fire(r + 1, (r + 1) % 2)
        for h in cur:
            h.wait()
        gbuf = gbufs[slot]
        ACC_UNROLL = 4

        def body(m, carry, gbuf=gbuf):
            accs = list(carry)
            l0 = m * ACC_UNROLL
            for d in range(ACC_UNROLL):
                v = gbuf[l0 + d, 0:PK]
                lo = plsc.bitcast(lax.shift_left(v, 16), jnp.float32)
                hi = plsc.bitcast(lax.bitwise_and(v, chi), jnp.float32)
                accs[2 * d] = accs[2 * d] + lo
                accs[2 * d + 1] = accs[2 * d + 1] + hi
            return tuple(accs)

        z = jnp.zeros((HALF,), jnp.float32)
        accs = lax.fori_loop(0, L // ACC_UNROLL, body, (z,) * (2 * ACC_UNROLL))
        acc_v[pl.ds(r * DIM, HALF)] = (accs[0] + accs[2]) + (accs[4] + accs[6])
        acc_v[pl.ds(r * DIM + HALF, HALF)] = (
            (accs[1] + accs[3]) + (accs[5] + accs[7]))

    pltpu.sync_copy(acc_v, out_hbm.at[pl.ds(wid * B_PER_W * DIM,
                                            B_PER_W * DIM)])
    gather_scope.__exit__(None, None, None)


def _gather_sum(embs_flat, reads_flat):
    mesh = plsc.VectorSubcoreMesh(core_axis_name="c", subcore_axis_name="s")
    fn = pl.kernel(
        _sc_body,
        out_type=jax.ShapeDtypeStruct((B * DIM,), jnp.float32),
        mesh=mesh,
        scratch_types=[
            pltpu.VMEM_SHARED((KMER_NUM, PK), jnp.int32),
            pltpu.VMEM((B_PER_W * L,), jnp.int32),
            pltpu.VMEM((PCW, EMB_COLS), jnp.float32),
            pltpu.VMEM((PCW, EMB_COLS), jnp.float32),
            pltpu.VMEM((PCL, PK), jnp.int32),
            pltpu.VMEM((PCL, PK), jnp.int32),
            pltpu.VMEM((L, PK), jnp.int32),
            pltpu.VMEM((L, PK), jnp.int32),
            pltpu.VMEM((B_PER_W * DIM,), jnp.float32),
            pltpu.SemaphoreType.DMA,
            pltpu.SemaphoreType.DMA,
            pltpu.SemaphoreType.DMA,
            pltpu.SemaphoreType.DMA,
            pltpu.SemaphoreType.DMA,
            pltpu.SemaphoreType.DMA,
        ],
        compiler_params=pltpu.CompilerParams(use_tc_tiling_on_sc=False,
                                             needs_layout_passes=False),
    )
    return fn(embs_flat, reads_flat)


def _loss_body(emb_ref, w_ref, lab_ref, out_ref):
    logits = lax.dot_general(
        emb_ref[...], w_ref[...],
        dimension_numbers=(((1,), (1,)), ((), ())),
        preferred_element_type=jnp.float32)
    m = jnp.max(logits, axis=1, keepdims=True)
    lse = m + jnp.log(jnp.sum(jnp.exp(logits - m), axis=1, keepdims=True))
    cls = lax.broadcasted_iota(jnp.int32, logits.shape, 1)
    picked = jnp.sum(jnp.where(cls == lab_ref[...], logits, 0.0),
                     axis=1, keepdims=True)
    out_ref[...] = jnp.sum(lse - picked, keepdims=True)


def _loss(read_emb, softmax_weights, read_labels):
    out = pl.pallas_call(
        _loss_body,
        out_shape=jax.ShapeDtypeStruct((1, 1), jnp.float32),
    )(read_emb, softmax_weights, read_labels.reshape(B, 1))
    return out[0, 0]


@jax.jit
def kernel(reads, read_labels, embs, softmax_weights):
    embs_v = embs.reshape(EMB_ROWS, EMB_COLS)
    read_emb_flat = _gather_sum(embs_v, reads.reshape(-1))
    read_emb = read_emb_flat.reshape(B, DIM)
    return _loss(read_emb, softmax_weights, read_labels)


# speedup vs baseline: 1.4456x; 1.4456x over previous
import jax
import jax.numpy as jnp
from jax import lax
from jax.experimental import pallas as pl
from jax.experimental.pallas import tpu as pltpu
from jax.experimental.pallas import tpu_sc as plsc

KMER_NUM = 100000
CLASS_NUM = 100
DIM = 32
B = 1024
L = 200

NC = 2
NS = 16
NW = NC * NS
B_PER_W = B // NW
IDX_COLS = 128
HALF = 16
PK = DIM // 2
KC = 160
N_CHUNKS_TOTAL = KMER_NUM // KC
NP_FULL = N_CHUNKS_TOTAL // NS


def _sc_body(embs_hbm, idx_hbm, out_hbm, tab_sh, idx_v,
             fbuf0, fbuf1, pbuf0, pbuf1, gbuf0, gbuf1, acc_v,
             psem0, psem1, gsem0, gsem1, osem0, osem1):
    cid = lax.axis_index("c")
    sid = lax.axis_index("s")
    wid = sid * NC + cid

    pltpu.sync_copy(idx_hbm.at[pl.ds(wid * B_PER_W * L, B_PER_W * L)], idx_v)

    fbufs, pbufs, psems = (fbuf0, fbuf1), (pbuf0, pbuf1), (psem0, psem1)
    osems = (osem0, osem1)
    c8000 = jnp.full((HALF,), 0x8000, jnp.int32)
    chi = jnp.full((HALF,), -65536, jnp.int32)
    kiota = lax.iota(jnp.int32, HALF)

    def pack_chunk(slot):
        fbuf, pbuf = fbufs[slot], pbufs[slot]

        def body(kb, carry):
            k0 = kb * HALF
            rows = k0 + kiota
            for j in range(PK):
                u1 = plsc.bitcast(fbuf[j, pl.ds(k0, HALF)], jnp.int32)
                u2 = plsc.bitcast(fbuf[j + HALF, pl.ds(k0, HALF)], jnp.int32)
                lo = lax.shift_right_logical(u1 + c8000, 16)
                hi = lax.bitwise_and(u2 + c8000, chi)
                plsc.store_scatter(
                    pbuf, [rows, jnp.full((HALF,), j, jnp.int32)],
                    lax.bitwise_or(lo, hi))
            return carry

        lax.fori_loop(0, KC // HALF, body, 0)

    def fire_in(m, slot):
        c = m * NS + sid
        return pltpu.async_copy(embs_hbm.at[:, pl.ds(c * KC, KC)],
                                fbufs[slot], psems[slot])

    def fire_out(m, slot):
        c = m * NS + sid
        return pltpu.async_copy(pbufs[slot], tab_sh.at[pl.ds(c * KC, KC)],
                                osems[slot])

    def wait_in(slot):
        pltpu.make_async_copy(embs_hbm.at[:, pl.ds(0, KC)],
                              fbufs[slot], psems[slot]).wait()

    def wait_out(slot):
        pltpu.make_async_copy(pbufs[slot], tab_sh.at[pl.ds(0, KC)],
                              osems[slot]).wait()

    with jax.named_scope("pack_phase"):
        fire_in(0, 0)
        fire_in(1, 1)

        def loop_body(it, carry):
            i2 = it * 2
            for b in range(2):
                m = i2 + b
                wait_in(b)
                pl.when(m >= 2)(lambda b=b: wait_out(b))
                pack_chunk(b)
                fire_out(m, b)
                if b == 0:
                    fire_in(m + 2, b)
                else:
                    def _prefetch(m=m, b=b):
                        fire_in(m + 2, b)
                    pl.when(m + 2 < NP_FULL)(_prefetch)
            return carry

        lax.fori_loop(0, (NP_FULL - 1) // 2, loop_body, 0)

        last = NP_FULL - 1
        wait_in(0)
        wait_out(0)
        pack_chunk(0)
        fire_out(last, 0)
        wait_out(1)
        wait_out(0)

        @pl.when(NP_FULL * NS + sid < N_CHUNKS_TOTAL)
        def _tail():
            c = NP_FULL * NS + sid
            pltpu.sync_copy(embs_hbm.at[:, pl.ds(c * KC, KC)], fbufs[0])
            pack_chunk(0)
            pltpu.sync_copy(pbufs[0], tab_sh.at[pl.ds(c * KC, KC)])

        plsc.subcore_barrier()

    gbufs, gsems = (gbuf0, gbuf1), (gsem0, gsem1)
    gather_scope = jax.named_scope("gather_phase")
    gather_scope.__enter__()

    def fire(r, slot):
        h0 = pltpu.async_copy(
            tab_sh.at[idx_v.at[pl.ds(r * L, IDX_COLS)]],
            gbufs[slot].at[pl.ds(0, IDX_COLS)], gsems[slot])
        h1 = pltpu.async_copy(
            tab_sh.at[idx_v.at[pl.ds(r * L + IDX_COLS, L - IDX_COLS)]],
            gbufs[slot].at[pl.ds(IDX_COLS, L - IDX_COLS)], gsems[slot])
        return [h0, h1]

    inflight = fire(0, 0)
    for r in range(B_PER_W):
        slot = r % 2
        cur = inflight
        if r + 1 < B_PER_W:
            inflight = fire(r + 1, (r + 1) % 2)
        for h in cur:
            h.wait()
        gbuf = gbufs[slot]
        ACC_UNROLL = 4

        def body(m, carry, gbuf=gbuf):
            accs = list(carry)
            l0 = m * ACC_UNROLL
            for d in range(ACC_UNROLL):
                v = gbuf[l0 + d, 0:PK]
                lo = plsc.bitcast(lax.shift_left(v, 16), jnp.float32)
                hi = plsc.bitcast(lax.bitwise_and(v, chi), jnp.float32)
                accs[2 * d] = accs[2 * d] + lo
                accs[2 * d + 1] = accs[2 * d + 1] + hi
            return tuple(accs)

        z = jnp.zeros((HALF,), jnp.float32)
        accs = lax.fori_loop(0, L // ACC_UNROLL, body, (z,) * (2 * ACC_UNROLL))
        acc_v[pl.ds(r * DIM, HALF)] = (accs[0] + accs[2]) + (accs[4] + accs[6])
        acc_v[pl.ds(r * DIM + HALF, HALF)] = (
            (accs[1] + accs[3]) + (accs[5] + accs[7]))

    pltpu.sync_copy(acc_v, out_hbm.at[pl.ds(wid * B_PER_W * DIM,
                                            B_PER_W * DIM)])
    gather_scope.__exit__(None, None, None)


def _gather_sum(embs_flat, reads_flat):
    mesh = plsc.VectorSubcoreMesh(core_axis_name="c", subcore_axis_name="s")
    fn = pl.kernel(
        _sc_body,
        out_type=jax.ShapeDtypeStruct((B * DIM,), jnp.float32),
        mesh=mesh,
        scratch_types=[
            pltpu.VMEM_SHARED((KMER_NUM, PK), jnp.int32),
            pltpu.VMEM((B_PER_W * L,), jnp.int32),
            pltpu.VMEM((DIM, KC), jnp.float32),
            pltpu.VMEM((DIM, KC), jnp.float32),
            pltpu.VMEM((KC, PK), jnp.int32),
            pltpu.VMEM((KC, PK), jnp.int32),
            pltpu.VMEM((L, PK), jnp.int32),
            pltpu.VMEM((L, PK), jnp.int32),
            pltpu.VMEM((B_PER_W * DIM,), jnp.float32),
            pltpu.SemaphoreType.DMA,
            pltpu.SemaphoreType.DMA,
            pltpu.SemaphoreType.DMA,
            pltpu.SemaphoreType.DMA,
            pltpu.SemaphoreType.DMA,
            pltpu.SemaphoreType.DMA,
        ],
        compiler_params=pltpu.CompilerParams(use_tc_tiling_on_sc=False,
                                             needs_layout_passes=False),
    )
    return fn(embs_flat, reads_flat)


def _loss_body(emb_ref, w_ref, lab_ref, out_ref):
    logits = lax.dot_general(
        emb_ref[...], w_ref[...],
        dimension_numbers=(((1,), (1,)), ((), ())),
        preferred_element_type=jnp.float32)
    m = jnp.max(logits, axis=1, keepdims=True)
    lse = m + jnp.log(jnp.sum(jnp.exp(logits - m), axis=1, keepdims=True))
    cls = lax.broadcasted_iota(jnp.int32, logits.shape, 1)
    picked = jnp.sum(jnp.where(cls == lab_ref[...], logits, 0.0),
                     axis=1, keepdims=True)
    out_ref[...] = jnp.sum(lse - picked, keepdims=True)


def _loss(read_emb, softmax_weights, read_labels):
    out = pl.pallas_call(
        _loss_body,
        out_shape=jax.ShapeDtypeStruct((1, 1), jnp.float32),
    )(read_emb, softmax_weights, read_labels.reshape(B, 1))
    return out[0, 0]


@jax.jit
def kernel(reads, read_labels, embs, softmax_weights):
    read_emb_flat = _gather_sum(embs.T, reads.reshape(-1))
    read_emb = read_emb_flat.reshape(B, DIM)
    return _loss(read_emb, softmax_weights, read_labels)


# speedup vs baseline: 1.7093x; 1.1825x over previous
import jax
import jax.numpy as jnp
from jax import lax
from jax.experimental import pallas as pl
from jax.experimental.pallas import tpu as pltpu
from jax.experimental.pallas import tpu_sc as plsc

KMER_NUM = 100000
CLASS_NUM = 100
DIM = 32
B = 1024
L = 200

NC = 2
NS = 16
NW = NC * NS
B_PER_W = B // NW
IDX_COLS = 128
HALF = 16
PK = DIM // 2
KC = 160
N_CHUNKS_TOTAL = KMER_NUM // KC
NP_FULL = N_CHUNKS_TOTAL // NS


def _sc_body(embs_hbm, idx_hbm, out_hbm, tab_sh, idx_v,
             fbuf0, fbuf1, pbuf0, pbuf1, gbuf0, gbuf1, acc_v,
             psem0, psem1, gsem0, gsem1, osem0, osem1):
    cid = lax.axis_index("c")
    sid = lax.axis_index("s")
    wid = sid * NC + cid

    pltpu.sync_copy(idx_hbm.at[pl.ds(wid * B_PER_W * L, B_PER_W * L)], idx_v)

    fbufs, pbufs, psems = (fbuf0, fbuf1), (pbuf0, pbuf1), (psem0, psem1)
    osems = (osem0, osem1)
    chi = jnp.full((HALF,), -65536, jnp.int32)
    kiota = lax.iota(jnp.int32, HALF)

    def pack_chunk(slot):
        fbuf, pbuf = fbufs[slot], pbufs[slot]

        def body(kb, carry):
            k0 = kb * HALF
            rows = k0 + kiota
            for j in range(PK):
                plsc.store_scatter(
                    pbuf, [rows, jnp.full((HALF,), j, jnp.int32)],
                    fbuf[j, pl.ds(k0, HALF)])
            return carry

        lax.fori_loop(0, KC // HALF, body, 0)

    def fire_in(m, slot):
        c = m * NS + sid
        return pltpu.async_copy(embs_hbm.at[:, pl.ds(c * KC, KC)],
                                fbufs[slot], psems[slot])

    def fire_out(m, slot):
        c = m * NS + sid
        return pltpu.async_copy(pbufs[slot], tab_sh.at[pl.ds(c * KC, KC)],
                                osems[slot])

    def wait_in(slot):
        pltpu.make_async_copy(embs_hbm.at[:, pl.ds(0, KC)],
                              fbufs[slot], psems[slot]).wait()

    def wait_out(slot):
        pltpu.make_async_copy(pbufs[slot], tab_sh.at[pl.ds(0, KC)],
                              osems[slot]).wait()

    with jax.named_scope("pack_phase"):
        fire_in(0, 0)
        fire_in(1, 1)

        def loop_body(it, carry):
            i2 = it * 2
            for b in range(2):
                m = i2 + b
                wait_in(b)
                pl.when(m >= 2)(lambda b=b: wait_out(b))
                pack_chunk(b)
                fire_out(m, b)
                if b == 0:
                    fire_in(m + 2, b)
                else:
                    def _prefetch(m=m, b=b):
                        fire_in(m + 2, b)
                    pl.when(m + 2 < NP_FULL)(_prefetch)
            return carry

        lax.fori_loop(0, (NP_FULL - 1) // 2, loop_body, 0)

        last = NP_FULL - 1
        wait_in(0)
        wait_out(0)
        pack_chunk(0)
        fire_out(last, 0)
        wait_out(1)
        wait_out(0)

        @pl.when(NP_FULL * NS + sid < N_CHUNKS_TOTAL)
        def _tail():
            c = NP_FULL * NS + sid
            pltpu.sync_copy(embs_hbm.at[:, pl.ds(c * KC, KC)], fbufs[0])
            pack_chunk(0)
            pltpu.sync_copy(pbufs[0], tab_sh.at[pl.ds(c * KC, KC)])

        plsc.subcore_barrier()

    gbufs, gsems = (gbuf0, gbuf1), (gsem0, gsem1)
    gather_scope = jax.named_scope("gather_phase")
    gather_scope.__enter__()

    def fire(r, slot):
        h0 = pltpu.async_copy(
            tab_sh.at[idx_v.at[pl.ds(r * L, IDX_COLS)]],
            gbufs[slot].at[pl.ds(0, IDX_COLS)], gsems[slot])
        h1 = pltpu.async_copy(
            tab_sh.at[idx_v.at[pl.ds(r * L + IDX_COLS, L - IDX_COLS)]],
            gbufs[slot].at[pl.ds(IDX_COLS, L - IDX_COLS)], gsems[slot])
        return [h0, h1]

    inflight = fire(0, 0)
    for r in range(B_PER_W):
        slot = r % 2
        cur = inflight
        if r + 1 < B_PER_W:
            inflight = fire(r + 1, (r + 1) % 2)
        for h in cur:
            h.wait()
        gbuf = gbufs[slot]
        ACC_UNROLL = 4

        def body(m, carry, gbuf=gbuf):
            accs = list(carry)
            l0 = m * ACC_UNROLL
            for d in range(ACC_UNROLL):
                v = gbuf[l0 + d, 0:PK]
                lo = plsc.bitcast(lax.shift_left(v, 16), jnp.float32)
                hi = plsc.bitcast(lax.bitwise_and(v, chi), jnp.float32)
                accs[2 * d] = accs[2 * d] + lo
                accs[2 * d + 1] = accs[2 * d + 1] + hi
            return tuple(accs)

        z = jnp.zeros((HALF,), jnp.float32)
        accs = lax.fori_loop(0, L // ACC_UNROLL, body, (z,) * (2 * ACC_UNROLL))
        acc_v[pl.ds(r * DIM, HALF)] = (accs[0] + accs[2]) + (accs[4] + accs[6])
        acc_v[pl.ds(r * DIM + HALF, HALF)] = (
            (accs[1] + accs[3]) + (accs[5] + accs[7]))

    pltpu.sync_copy(acc_v, out_hbm.at[pl.ds(wid * B_PER_W * DIM,
                                            B_PER_W * DIM)])
    gather_scope.__exit__(None, None, None)


def _gather_sum(embs_flat, reads_flat):
    mesh = plsc.VectorSubcoreMesh(core_axis_name="c", subcore_axis_name="s")
    fn = pl.kernel(
        _sc_body,
        out_type=jax.ShapeDtypeStruct((B * DIM,), jnp.float32),
        mesh=mesh,
        scratch_types=[
            pltpu.VMEM_SHARED((KMER_NUM, PK), jnp.int32),
            pltpu.VMEM((B_PER_W * L,), jnp.int32),
            pltpu.VMEM((PK, KC), jnp.int32),
            pltpu.VMEM((PK, KC), jnp.int32),
            pltpu.VMEM((KC, PK), jnp.int32),
            pltpu.VMEM((KC, PK), jnp.int32),
            pltpu.VMEM((L, PK), jnp.int32),
            pltpu.VMEM((L, PK), jnp.int32),
            pltpu.VMEM((B_PER_W * DIM,), jnp.float32),
            pltpu.SemaphoreType.DMA,
            pltpu.SemaphoreType.DMA,
            pltpu.SemaphoreType.DMA,
            pltpu.SemaphoreType.DMA,
            pltpu.SemaphoreType.DMA,
            pltpu.SemaphoreType.DMA,
        ],
        compiler_params=pltpu.CompilerParams(use_tc_tiling_on_sc=False,
                                             needs_layout_passes=False),
    )
    return fn(embs_flat, reads_flat)


def _pack_body(x_ref, o_ref):
    u = lax.bitcast_convert_type(x_ref[...], jnp.int32) + 0x8000
    lo = lax.shift_right_logical(u[0:HALF, :], 16)
    hi = lax.bitwise_and(u[HALF:DIM, :], -65536)
    o_ref[...] = lax.bitwise_or(lo, hi)


def _tc_pack(embs_t):
    return pl.pallas_call(
        _pack_body,
        out_shape=jax.ShapeDtypeStruct((PK, KMER_NUM), jnp.int32),
    )(embs_t)


def _loss_body(emb_ref, w_ref, lab_ref, out_ref):
    logits = lax.dot_general(
        emb_ref[...], w_ref[...],
        dimension_numbers=(((1,), (1,)), ((), ())),
        preferred_element_type=jnp.float32)
    m = jnp.max(logits, axis=1, keepdims=True)
    lse = m + jnp.log(jnp.sum(jnp.exp(logits - m), axis=1, keepdims=True))
    cls = lax.broadcasted_iota(jnp.int32, logits.shape, 1)
    picked = jnp.sum(jnp.where(cls == lab_ref[...], logits, 0.0),
                     axis=1, keepdims=True)
    out_ref[...] = jnp.sum(lse - picked, keepdims=True)


def _loss(read_emb, softmax_weights, read_labels):
    out = pl.pallas_call(
        _loss_body,
        out_shape=jax.ShapeDtypeStruct((1, 1), jnp.float32),
    )(read_emb, softmax_weights, read_labels.reshape(B, 1))
    return out[0, 0]


@jax.jit
def kernel(reads, read_labels, embs, softmax_weights):
    packed_t = _tc_pack(embs.T)
    read_emb_flat = _gather_sum(packed_t, reads.reshape(-1))
    read_emb = read_emb_flat.reshape(B, DIM)
    return _loss(read_emb, softmax_weights, read_labels)


# speedup vs baseline: 1.7514x; 1.0246x over previous
import jax
import jax.numpy as jnp
from jax import lax
from jax.experimental import pallas as pl
from jax.experimental.pallas import tpu as pltpu
from jax.experimental.pallas import tpu_sc as plsc

KMER_NUM = 100000
CLASS_NUM = 100
DIM = 32
B = 1024
L = 200

NC = 2
NS = 16
NW = NC * NS
B_PER_W = B // NW
IDX_COLS = 128
HALF = 16
PK = DIM // 2
KC = 160
N_CHUNKS_TOTAL = KMER_NUM // KC
NP_FULL = N_CHUNKS_TOTAL // NS


def _sc_body(embs_hbm, idx_hbm, out_hbm, tab_sh, idx_v,
             fbuf0, fbuf1, fbuf2, fbuf3, pbuf0, pbuf1, gbuf0, gbuf1, acc_v,
             psem0, psem1, psem2, psem3, gsem0, gsem1, osem0, osem1):
    cid = lax.axis_index("c")
    sid = lax.axis_index("s")
    wid = sid * NC + cid

    pltpu.sync_copy(idx_hbm.at[pl.ds(wid * B_PER_W * L, B_PER_W * L)], idx_v)

    fbufs = (fbuf0, fbuf1, fbuf2, fbuf3)
    psems = (psem0, psem1, psem2, psem3)
    pbufs, osems = (pbuf0, pbuf1), (osem0, osem1)
    NBUF = 4
    chi = jnp.full((HALF,), -65536, jnp.int32)
    kiota = lax.iota(jnp.int32, HALF)

    def pack_chunk(fslot, oslot):
        fbuf, pbuf = fbufs[fslot], pbufs[oslot]

        def body(kb, carry):
            k0 = kb * HALF
            rows = k0 + kiota
            for j in range(PK):
                plsc.store_scatter(
                    pbuf, [rows, jnp.full((HALF,), j, jnp.int32)],
                    fbuf[j, pl.ds(k0, HALF)])
            return carry

        lax.fori_loop(0, KC // HALF, body, 0)

    def fire_in(m, slot):
        c = m * NS + sid
        return pltpu.async_copy(embs_hbm.at[:, pl.ds(c * KC, KC)],
                                fbufs[slot], psems[slot])

    def fire_out(m, slot):
        c = m * NS + sid
        return pltpu.async_copy(pbufs[slot], tab_sh.at[pl.ds(c * KC, KC)],
                                osems[slot])

    def wait_in(slot):
        pltpu.make_async_copy(embs_hbm.at[:, pl.ds(0, KC)],
                              fbufs[slot], psems[slot]).wait()

    def wait_out(slot):
        pltpu.make_async_copy(pbufs[slot], tab_sh.at[pl.ds(0, KC)],
                              osems[slot]).wait()

    with jax.named_scope("pack_phase"):
        for b in range(NBUF):
            fire_in(b, b)

        def loop_body(it, carry):
            i4 = it * 4
            for b in range(NBUF):
                m = i4 + b
                wait_in(b)
                pl.when(m >= 2)(lambda b=b: wait_out(b % 2))
                pack_chunk(b, b % 2)
                fire_out(m, b % 2)

                def _prefetch(m=m, b=b):
                    fire_in(m + NBUF, b)
                if b < 3:
                    _prefetch()
                else:
                    pl.when(m + NBUF < NP_FULL)(_prefetch)
            return carry

        lax.fori_loop(0, (NP_FULL - 3) // 4, loop_body, 0)

        for m in range(NP_FULL - 3, NP_FULL):
            b = m % NBUF
            wait_in(b)
            wait_out(m % 2)
            pack_chunk(b, m % 2)
            fire_out(m, m % 2)
        wait_out(0)
        wait_out(1)

        @pl.when(NP_FULL * NS + sid < N_CHUNKS_TOTAL)
        def _tail():
            c = NP_FULL * NS + sid
            pltpu.sync_copy(embs_hbm.at[:, pl.ds(c * KC, KC)], fbufs[0])
            pack_chunk(0, 0)
            pltpu.sync_copy(pbufs[0], tab_sh.at[pl.ds(c * KC, KC)])

        plsc.subcore_barrier()

    gbufs, gsems = (gbuf0, gbuf1), (gsem0, gsem1)
    gather_scope = jax.named_scope("gather_phase")
    gather_scope.__enter__()

    def fire(r, slot):
        h0 = pltpu.async_copy(
            tab_sh.at[idx_v.at[pl.ds(r * L, IDX_COLS)]],
            gbufs[slot].at[pl.ds(0, IDX_COLS)], gsems[slot])
        h1 = pltpu.async_copy(
            tab_sh.at[idx_v.at[pl.ds(r * L + IDX_COLS, L - IDX_COLS)]],
            gbufs[slot].at[pl.ds(IDX_COLS, L - IDX_COLS)], gsems[slot])
        return [h0, h1]

    inflight = fire(0, 0)
    for r in range(B_PER_W):
        slot = r % 2
        cur = inflight
        if r + 1 < B_PER_W:
            inflight = fire(r + 1, (r + 1) % 2)
        for h in cur:
            h.wait()
        gbuf = gbufs[slot]
        ACC_UNROLL = 4

        def body(m, carry, gbuf=gbuf):
            accs = list(carry)
            l0 = m * ACC_UNROLL
            for d in range(ACC_UNROLL):
                v = gbuf[l0 + d, 0:PK]
                lo = plsc.bitcast(lax.shift_left(v, 16), jnp.float32)
                hi = plsc.bitcast(lax.bitwise_and(v, chi), jnp.float32)
                accs[2 * d] = accs[2 * d] + lo
                accs[2 * d + 1] = accs[2 * d + 1] + hi
            return tuple(accs)

        z = jnp.zeros((HALF,), jnp.float32)
        accs = lax.fori_loop(0, L // ACC_UNROLL, body, (z,) * (2 * ACC_UNROLL))
        acc_v[pl.ds(r * DIM, HALF)] = (accs[0] + accs[2]) + (accs[4] + accs[6])
        acc_v[pl.ds(r * DIM + HALF, HALF)] = (
            (accs[1] + accs[3]) + (accs[5] + accs[7]))

    pltpu.sync_copy(acc_v, out_hbm.at[pl.ds(wid * B_PER_W * DIM,
                                            B_PER_W * DIM)])
    gather_scope.__exit__(None, None, None)


def _gather_sum(embs_flat, reads_flat):
    mesh = plsc.VectorSubcoreMesh(core_axis_name="c", subcore_axis_name="s")
    fn = pl.kernel(
        _sc_body,
        out_type=jax.ShapeDtypeStruct((B * DIM,), jnp.float32),
        mesh=mesh,
        scratch_types=[
            pltpu.VMEM_SHARED((KMER_NUM, PK), jnp.int32),
            pltpu.VMEM((B_PER_W * L,), jnp.int32),
            pltpu.VMEM((PK, KC), jnp.int32),
            pltpu.VMEM((PK, KC), jnp.int32),
            pltpu.VMEM((PK, KC), jnp.int32),
            pltpu.VMEM((PK, KC), jnp.int32),
            pltpu.VMEM((KC, PK), jnp.int32),
            pltpu.VMEM((KC, PK), jnp.int32),
            pltpu.VMEM((L, PK), jnp.int32),
            pltpu.VMEM((L, PK), jnp.int32),
            pltpu.VMEM((B_PER_W * DIM,), jnp.float32),
            pltpu.SemaphoreType.DMA,
            pltpu.SemaphoreType.DMA,
            pltpu.SemaphoreType.DMA,
            pltpu.SemaphoreType.DMA,
            pltpu.SemaphoreType.DMA,
            pltpu.SemaphoreType.DMA,
            pltpu.SemaphoreType.DMA,
            pltpu.SemaphoreType.DMA,
        ],
        compiler_params=pltpu.CompilerParams(use_tc_tiling_on_sc=False,
                                             needs_layout_passes=False),
    )
    return fn(embs_flat, reads_flat)


def _pack_body(x_ref, o_ref):
    u = lax.bitcast_convert_type(x_ref[...], jnp.int32) + 0x8000
    lo = lax.shift_right_logical(u[0:HALF, :], 16)
    hi = lax.bitwise_and(u[HALF:DIM, :], -65536)
    o_ref[...] = lax.bitwise_or(lo, hi)


def _tc_pack(embs_t):
    return pl.pallas_call(
        _pack_body,
        out_shape=jax.ShapeDtypeStruct((PK, KMER_NUM), jnp.int32),
    )(embs_t)


def _loss_body(emb_ref, w_ref, lab_ref, out_ref):
    logits = lax.dot_general(
        emb_ref[...], w_ref[...],
        dimension_numbers=(((1,), (1,)), ((), ())),
        preferred_element_type=jnp.float32)
    m = jnp.max(logits, axis=1, keepdims=True)
    lse = m + jnp.log(jnp.sum(jnp.exp(logits - m), axis=1, keepdims=True))
    cls = lax.broadcasted_iota(jnp.int32, logits.shape, 1)
    picked = jnp.sum(jnp.where(cls == lab_ref[...], logits, 0.0),
                     axis=1, keepdims=True)
    out_ref[...] = jnp.sum(lse - picked, keepdims=True)


def _loss(read_emb, softmax_weights, read_labels):
    out = pl.pallas_call(
        _loss_body,
        out_shape=jax.ShapeDtypeStruct((1, 1), jnp.float32),
    )(read_emb, softmax_weights, read_labels.reshape(B, 1))
    return out[0, 0]


@jax.jit
def kernel(reads, read_labels, embs, softmax_weights):
    packed_t = _tc_pack(embs.T)
    read_emb_flat = _gather_sum(packed_t, reads.reshape(-1))
    read_emb = read_emb_flat.reshape(B, DIM)
    return _loss(read_emb, softmax_weights, read_labels)
